# baseline (speedup 1.0000x reference)
import functools

import jax
import jax.numpy as jnp
from jax import lax
from jax.experimental import pallas as pl
from jax.experimental.pallas import tpu as pltpu
from jax.experimental.pallas import tpu_sc as plsc

NC = 2
NS = 16
NW = NC * NS

BATCH = 1024
NEG = 200
DIM = 64
SLOTS = NEG + 1
MAX_SLOTS_W = 13
Q = BATCH // 4
IDXC = 128


def _transpose_quarter(gb, tb, iota16):
  def per_group(g, carry):
    rows = g * 16 + iota16

    for d in range(DIM):
      v = plsc.load_gather(gb, [rows, jnp.full((16,), d, jnp.int32)])
      tb[d, pl.ds(g * 16, 16)] = v
    return carry

  lax.fori_loop(0, Q // 16, per_group, 0)


def _body(idx_hbm, oe_hbm, orl_hbm, e_hbm, rl_hbm,
          out_ot, out_oh, out_orl, out_t, out_h, out_rl,
          idx_v, sidx_v, gbuf, tbuf, sem_g, sem_o):
  wid = lax.axis_index("s") * NC + lax.axis_index("c")
  parity = lax.rem(wid, 2)
  s0 = lax.div(wid, 2)
  iota16 = lax.iota(jnp.int32, 16)

  for jj in range(MAX_SLOTS_W):
    slot = s0 + 16 * jj

    @pl.when(slot < SLOTS)
    def _stage():
      pltpu.sync_copy(idx_hbm.at[2 + slot], idx_v.at[jj])

  n_q = 48 + 4 * (s0 <= 8).astype(jnp.int32)

  def entity_loop(tbl, out3):
    def fire(t):
      jj, q = lax.div(t, 4), lax.rem(t, 4)
      g = lax.rem(t, 2)
      base = q * Q
      pltpu.async_copy(tbl.at[idx_v.at[jj, pl.ds(base, IDXC)]],
                       gbuf.at[g, pl.ds(0, IDXC)], sem_g)
      pltpu.async_copy(tbl.at[idx_v.at[jj, pl.ds(base + IDXC, IDXC)]],
                       gbuf.at[g, pl.ds(IDXC, IDXC)], sem_g)

    def wait_gather(t):
      jj, q = lax.div(t, 4), lax.rem(t, 4)
      g = lax.rem(t, 2)
      base = q * Q
      pltpu.make_async_copy(tbl.at[idx_v.at[jj, pl.ds(base, IDXC)]],
                            gbuf.at[g, pl.ds(0, IDXC)], sem_g).wait()
      pltpu.make_async_copy(tbl.at[idx_v.at[jj, pl.ds(base + IDXC, IDXC)]],
                            gbuf.at[g, pl.ds(IDXC, IDXC)], sem_g).wait()

    fire(0)

    def step(t, carry):
      jj, q = lax.div(t, 4), lax.rem(t, 4)
      slot = s0 + 16 * jj
      g = lax.rem(t, 2)
      wait_gather(t)

      @pl.when(t + 1 < n_q)
      def _fire_next():
        fire(t + 1)

      @pl.when(t >= 2)
      def _drain_out():
        pltpu.make_async_copy(tbuf.at[0], out3.at[0, :, pl.ds(0, Q)],
                              sem_o).wait()

      _transpose_quarter(gbuf.at[g], tbuf.at[g], iota16)
      pltpu.async_copy(tbuf.at[g], out3.at[slot, :, pl.ds(q * Q, Q)], sem_o)
      return carry

    lax.fori_loop(0, n_q, step, 0)
    for _ in range(2):
      pltpu.make_async_copy(tbuf.at[0], out3.at[0, :, pl.ds(0, Q)],
                            sem_o).wait()

  @pl.when(parity == 0)
  def _even():
    entity_loop(oe_hbm, out_ot)

  @pl.when(parity == 1)
  def _odd():
    entity_loop(e_hbm, out_t)

  def small_job(tbl, out2, idx_row):
    pltpu.sync_copy(idx_hbm.at[idx_row], sidx_v)

    def sq(q, carry):
      base = q * Q
      c1 = pltpu.async_copy(tbl.at[sidx_v.at[pl.ds(base, IDXC)]],
                            gbuf.at[0, pl.ds(0, IDXC)], sem_g)
      c2 = pltpu.async_copy(tbl.at[sidx_v.at[pl.ds(base + IDXC, IDXC)]],
                            gbuf.at[0, pl.ds(IDXC, IDXC)], sem_g)
      c1.wait()
      c2.wait()
      _transpose_quarter(gbuf.at[0], tbuf.at[0], iota16)
      pltpu.sync_copy(tbuf.at[0], out2.at[:, pl.ds(base, Q)])
      return carry

    lax.fori_loop(0, 4, sq, 0)

  @pl.when(wid == 28)
  def _oh():
    small_job(oe_hbm, out_oh, 0)

  @pl.when(wid == 29)
  def _h():
    small_job(e_hbm, out_h, 0)

  @pl.when(wid == 30)
  def _orl():
    small_job(orl_hbm, out_orl, 1)

  @pl.when(wid == 31)
  def _rl():
    small_job(rl_hbm, out_rl, 1)


@jax.jit
def _gather_all(idx_all, oe, orl, e, rl):
  mesh = plsc.VectorSubcoreMesh(core_axis_name="c", subcore_axis_name="s")
  f32 = jnp.float32
  run = functools.partial(
      pl.kernel,
      out_type=(
          jax.ShapeDtypeStruct((SLOTS, DIM, BATCH), f32),
          jax.ShapeDtypeStruct((DIM, BATCH), f32),
          jax.ShapeDtypeStruct((DIM, BATCH), f32),
          jax.ShapeDtypeStruct((SLOTS, DIM, BATCH), f32),
          jax.ShapeDtypeStruct((DIM, BATCH), f32),
          jax.ShapeDtypeStruct((DIM, BATCH), f32),
      ),
      mesh=mesh,
      compiler_params=pltpu.CompilerParams(use_tc_tiling_on_sc=False,
                                           needs_layout_passes=False),
      scratch_types=[
          pltpu.VMEM((MAX_SLOTS_W, BATCH), jnp.int32),
          pltpu.VMEM((BATCH,), jnp.int32),
          pltpu.VMEM((2, Q, DIM), f32),
          pltpu.VMEM((2, DIM, Q), f32),
          pltpu.SemaphoreType.DMA,
          pltpu.SemaphoreType.DMA,
      ],
  )(_body)
  return run(idx_all, oe, orl, e, rl)


def kernel(positive, negative, origin_entity_embedding,
           origin_relation_embedding, entity_embedding, relation_embedding):
  idx_all = jnp.concatenate([positive.T, negative.T], axis=0).astype(jnp.int32)

  ot_p, oh_p, orl_p, t_p, h_p, rl_p = _gather_all(
      idx_all, origin_entity_embedding, origin_relation_embedding,
      entity_embedding, relation_embedding)

  return (oh_p.T.reshape(BATCH, 1, DIM),
          orl_p.T.reshape(BATCH, 1, DIM),
          ot_p.transpose(2, 0, 1),
          h_p.T.reshape(BATCH, 1, DIM),
          rl_p.T.reshape(BATCH, 1, DIM),
          t_p.transpose(2, 0, 1))


# speedup vs baseline: 1.4381x; 1.4381x over previous
import functools

import jax
import jax.numpy as jnp
from jax import lax
from jax.experimental import pallas as pl
from jax.experimental.pallas import tpu as pltpu
from jax.experimental.pallas import tpu_sc as plsc

NC = 2
NS = 16
NW = NC * NS

BATCH = 1024
NEG = 200
DIM = 64
SLOTS = NEG + 1
MAX_SLOTS_W = 13
Q = BATCH // 4
IDXC = 128


def _transpose_quarter(gb, tb, iota16):
  def per_group(g, carry):
    rows = g * 16 + iota16

    for dc in range(0, DIM, 16):
      vs = [plsc.load_gather(gb, [rows, jnp.full((16,), d, jnp.int32)])
            for d in range(dc, dc + 16)]
      for k, d in enumerate(range(dc, dc + 16)):
        tb[d, pl.ds(g * 16, 16)] = vs[k]
    return carry

  lax.fori_loop(0, Q // 16, per_group, 0)


def _body(idx_hbm, oe_hbm, orl_hbm, e_hbm, rl_hbm,
          out_ot, out_oh, out_orl, out_t, out_h, out_rl,
          idx_v, sidx_v, gbuf, tbuf, sem_g, sem_o):
  wid = lax.axis_index("s") * NC + lax.axis_index("c")
  parity = lax.rem(wid, 2)
  s0 = lax.div(wid, 2)
  iota16 = lax.iota(jnp.int32, 16)

  for jj in range(MAX_SLOTS_W):
    slot = s0 + 16 * jj

    @pl.when(slot < SLOTS)
    def _stage():
      pltpu.sync_copy(idx_hbm.at[2 + slot], idx_v.at[jj])

  n_q = 48 + 4 * (s0 <= 8).astype(jnp.int32)

  def entity_loop(tbl, out3):
    def fire(t):
      jj, q = lax.div(t, 4), lax.rem(t, 4)
      g = lax.rem(t, 2)
      base = q * Q
      pltpu.async_copy(tbl.at[idx_v.at[jj, pl.ds(base, IDXC)]],
                       gbuf.at[g, pl.ds(0, IDXC)], sem_g)
      pltpu.async_copy(tbl.at[idx_v.at[jj, pl.ds(base + IDXC, IDXC)]],
                       gbuf.at[g, pl.ds(IDXC, IDXC)], sem_g)

    def wait_gather(t):
      jj, q = lax.div(t, 4), lax.rem(t, 4)
      g = lax.rem(t, 2)
      base = q * Q
      pltpu.make_async_copy(tbl.at[idx_v.at[jj, pl.ds(base, IDXC)]],
                            gbuf.at[g, pl.ds(0, IDXC)], sem_g).wait()
      pltpu.make_async_copy(tbl.at[idx_v.at[jj, pl.ds(base + IDXC, IDXC)]],
                            gbuf.at[g, pl.ds(IDXC, IDXC)], sem_g).wait()

    fire(0)

    def step(t, carry):
      jj, q = lax.div(t, 4), lax.rem(t, 4)
      slot = s0 + 16 * jj
      g = lax.rem(t, 2)
      wait_gather(t)

      @pl.when(t + 1 < n_q)
      def _fire_next():
        fire(t + 1)

      @pl.when(t >= 2)
      def _drain_out():
        pltpu.make_async_copy(tbuf.at[0], out3.at[0, :, pl.ds(0, Q)],
                              sem_o).wait()

      _transpose_quarter(gbuf.at[g], tbuf.at[g], iota16)
      pltpu.async_copy(tbuf.at[g], out3.at[slot, :, pl.ds(q * Q, Q)], sem_o)
      return carry

    lax.fori_loop(0, n_q, step, 0)
    for _ in range(2):
      pltpu.make_async_copy(tbuf.at[0], out3.at[0, :, pl.ds(0, Q)],
                            sem_o).wait()

  @pl.when(parity == 0)
  def _even():
    entity_loop(oe_hbm, out_ot)

  @pl.when(parity == 1)
  def _odd():
    entity_loop(e_hbm, out_t)

  def small_job(tbl, out2, idx_row):
    pltpu.sync_copy(idx_hbm.at[idx_row], sidx_v)

    def sq(q, carry):
      base = q * Q
      c1 = pltpu.async_copy(tbl.at[sidx_v.at[pl.ds(base, IDXC)]],
                            gbuf.at[0, pl.ds(0, IDXC)], sem_g)
      c2 = pltpu.async_copy(tbl.at[sidx_v.at[pl.ds(base + IDXC, IDXC)]],
                            gbuf.at[0, pl.ds(IDXC, IDXC)], sem_g)
      c1.wait()
      c2.wait()
      _transpose_quarter(gbuf.at[0], tbuf.at[0], iota16)
      pltpu.sync_copy(tbuf.at[0], out2.at[:, pl.ds(base, Q)])
      return carry

    lax.fori_loop(0, 4, sq, 0)

  @pl.when(wid == 28)
  def _oh():
    small_job(oe_hbm, out_oh, 0)

  @pl.when(wid == 29)
  def _h():
    small_job(e_hbm, out_h, 0)

  @pl.when(wid == 30)
  def _orl():
    small_job(orl_hbm, out_orl, 1)

  @pl.when(wid == 31)
  def _rl():
    small_job(rl_hbm, out_rl, 1)


@jax.jit
def _gather_all(idx_all, oe, orl, e, rl):
  mesh = plsc.VectorSubcoreMesh(core_axis_name="c", subcore_axis_name="s")
  f32 = jnp.float32
  run = functools.partial(
      pl.kernel,
      out_type=(
          jax.ShapeDtypeStruct((SLOTS, DIM, BATCH), f32),
          jax.ShapeDtypeStruct((DIM, BATCH), f32),
          jax.ShapeDtypeStruct((DIM, BATCH), f32),
          jax.ShapeDtypeStruct((SLOTS, DIM, BATCH), f32),
          jax.ShapeDtypeStruct((DIM, BATCH), f32),
          jax.ShapeDtypeStruct((DIM, BATCH), f32),
      ),
      mesh=mesh,
      compiler_params=pltpu.CompilerParams(use_tc_tiling_on_sc=False,
                                           needs_layout_passes=False),
      scratch_types=[
          pltpu.VMEM((MAX_SLOTS_W, BATCH), jnp.int32),
          pltpu.VMEM((BATCH,), jnp.int32),
          pltpu.VMEM((2, Q, DIM), f32),
          pltpu.VMEM((2, DIM, Q), f32),
          pltpu.SemaphoreType.DMA,
          pltpu.SemaphoreType.DMA,
      ],
  )(_body)
  return run(idx_all, oe, orl, e, rl)


def kernel(positive, negative, origin_entity_embedding,
           origin_relation_embedding, entity_embedding, relation_embedding):
  idx_all = jnp.concatenate([positive.T, negative.T], axis=0).astype(jnp.int32)

  ot_p, oh_p, orl_p, t_p, h_p, rl_p = _gather_all(
      idx_all, origin_entity_embedding, origin_relation_embedding,
      entity_embedding, relation_embedding)

  return (oh_p.T.reshape(BATCH, 1, DIM),
          orl_p.T.reshape(BATCH, 1, DIM),
          ot_p.transpose(2, 0, 1),
          h_p.T.reshape(BATCH, 1, DIM),
          rl_p.T.reshape(BATCH, 1, DIM),
          t_p.transpose(2, 0, 1))


# speedup vs baseline: 1.4389x; 1.0005x over previous
import functools

import jax
import jax.numpy as jnp
from jax import lax
from jax.experimental import pallas as pl
from jax.experimental.pallas import tpu as pltpu
from jax.experimental.pallas import tpu_sc as plsc

NC = 2
NS = 16
NW = NC * NS

BATCH = 1024
NEG = 200
DIM = 64
SLOTS = NEG + 1
MAX_SLOTS_W = 13
Q = BATCH // 4
IDXC = 128


def _transpose_quarter(gb, tb, iota16):
  def per_group(g, carry):
    rows = g * 16 + iota16

    for dc in range(0, DIM, 16):
      vs = [plsc.load_gather(gb, [rows, jnp.full((16,), d, jnp.int32)])
            for d in range(dc, dc + 16)]
      for k, d in enumerate(range(dc, dc + 16)):
        tb[d, pl.ds(g * 16, 16)] = vs[k]
    return carry

  lax.fori_loop(0, Q // 16, per_group, 0)


def _body(idx_hbm, oe_hbm, orl_hbm, e_hbm, rl_hbm,
          out_ot, out_oh, out_orl, out_t, out_h, out_rl,
          idx_v, sidx_v, gbuf, tbuf, sem_g, sem_o):
  wid = lax.axis_index("s") * NC + lax.axis_index("c")
  parity = lax.rem(wid, 2)
  s0 = lax.div(wid, 2)
  iota16 = lax.iota(jnp.int32, 16)

  for jj in range(MAX_SLOTS_W):
    slot = s0 + 16 * jj

    @pl.when(slot < SLOTS)
    def _stage():
      pltpu.sync_copy(idx_hbm.at[2 + slot], idx_v.at[jj])

  n_q = 48 + 4 * (s0 <= 8).astype(jnp.int32)

  def entity_loop(tbl, out3):
    def fire(t):
      jj, q = lax.div(t, 4), lax.rem(t, 4)
      g = lax.rem(t, 3)
      base = q * Q
      pltpu.async_copy(tbl.at[idx_v.at[jj, pl.ds(base, IDXC)]],
                       gbuf.at[g, pl.ds(0, IDXC)], sem_g)
      pltpu.async_copy(tbl.at[idx_v.at[jj, pl.ds(base + IDXC, IDXC)]],
                       gbuf.at[g, pl.ds(IDXC, IDXC)], sem_g)

    def wait_gather(t):
      jj, q = lax.div(t, 4), lax.rem(t, 4)
      g = lax.rem(t, 3)
      base = q * Q
      pltpu.make_async_copy(tbl.at[idx_v.at[jj, pl.ds(base, IDXC)]],
                            gbuf.at[g, pl.ds(0, IDXC)], sem_g).wait()
      pltpu.make_async_copy(tbl.at[idx_v.at[jj, pl.ds(base + IDXC, IDXC)]],
                            gbuf.at[g, pl.ds(IDXC, IDXC)], sem_g).wait()

    fire(0)
    fire(1)

    def step(t, carry):
      jj, q = lax.div(t, 4), lax.rem(t, 4)
      slot = s0 + 16 * jj
      g = lax.rem(t, 3)
      wait_gather(t)

      @pl.when(t + 2 < n_q)
      def _fire_next():
        fire(t + 2)

      @pl.when(t >= 3)
      def _drain_out():
        pltpu.make_async_copy(tbuf.at[0], out3.at[0, :, pl.ds(0, Q)],
                              sem_o).wait()

      _transpose_quarter(gbuf.at[g], tbuf.at[g], iota16)
      pltpu.async_copy(tbuf.at[g], out3.at[slot, :, pl.ds(q * Q, Q)], sem_o)
      return carry

    lax.fori_loop(0, n_q, step, 0)
    for _ in range(3):
      pltpu.make_async_copy(tbuf.at[0], out3.at[0, :, pl.ds(0, Q)],
                            sem_o).wait()

  @pl.when(parity == 0)
  def _even():
    entity_loop(oe_hbm, out_ot)

  @pl.when(parity == 1)
  def _odd():
    entity_loop(e_hbm, out_t)

  def small_job(tbl, out2, idx_row):
    pltpu.sync_copy(idx_hbm.at[idx_row], sidx_v)

    def sq(q, carry):
      base = q * Q
      c1 = pltpu.async_copy(tbl.at[sidx_v.at[pl.ds(base, IDXC)]],
                            gbuf.at[0, pl.ds(0, IDXC)], sem_g)
      c2 = pltpu.async_copy(tbl.at[sidx_v.at[pl.ds(base + IDXC, IDXC)]],
                            gbuf.at[0, pl.ds(IDXC, IDXC)], sem_g)
      c1.wait()
      c2.wait()
      _transpose_quarter(gbuf.at[0], tbuf.at[0], iota16)
      pltpu.sync_copy(tbuf.at[0], out2.at[:, pl.ds(base, Q)])
      return carry

    lax.fori_loop(0, 4, sq, 0)

  @pl.when(wid == 28)
  def _oh():
    small_job(oe_hbm, out_oh, 0)

  @pl.when(wid == 29)
  def _h():
    small_job(e_hbm, out_h, 0)

  @pl.when(wid == 30)
  def _orl():
    small_job(orl_hbm, out_orl, 1)

  @pl.when(wid == 31)
  def _rl():
    small_job(rl_hbm, out_rl, 1)


@jax.jit
def _gather_all(idx_all, oe, orl, e, rl):
  mesh = plsc.VectorSubcoreMesh(core_axis_name="c", subcore_axis_name="s")
  f32 = jnp.float32
  run = functools.partial(
      pl.kernel,
      out_type=(
          jax.ShapeDtypeStruct((SLOTS, DIM, BATCH), f32),
          jax.ShapeDtypeStruct((DIM, BATCH), f32),
          jax.ShapeDtypeStruct((DIM, BATCH), f32),
          jax.ShapeDtypeStruct((SLOTS, DIM, BATCH), f32),
          jax.ShapeDtypeStruct((DIM, BATCH), f32),
          jax.ShapeDtypeStruct((DIM, BATCH), f32),
      ),
      mesh=mesh,
      compiler_params=pltpu.CompilerParams(use_tc_tiling_on_sc=False,
                                           needs_layout_passes=False),
      scratch_types=[
          pltpu.VMEM((MAX_SLOTS_W, BATCH), jnp.int32),
          pltpu.VMEM((BATCH,), jnp.int32),
          pltpu.VMEM((3, Q, DIM), f32),
          pltpu.VMEM((3, DIM, Q), f32),
          pltpu.SemaphoreType.DMA,
          pltpu.SemaphoreType.DMA,
      ],
  )(_body)
  return run(idx_all, oe, orl, e, rl)


def kernel(positive, negative, origin_entity_embedding,
           origin_relation_embedding, entity_embedding, relation_embedding):
  idx_all = jnp.concatenate([positive.T, negative.T], axis=0).astype(jnp.int32)

  ot_p, oh_p, orl_p, t_p, h_p, rl_p = _gather_all(
      idx_all, origin_entity_embedding, origin_relation_embedding,
      entity_embedding, relation_embedding)

  return (oh_p.T.reshape(BATCH, 1, DIM),
          orl_p.T.reshape(BATCH, 1, DIM),
          ot_p.transpose(2, 0, 1),
          h_p.T.reshape(BATCH, 1, DIM),
          rl_p.T.reshape(BATCH, 1, DIM),
          t_p.transpose(2, 0, 1))


# speedup vs baseline: 2.2939x; 1.5942x over previous
import functools

import jax
import jax.numpy as jnp
from jax import lax
from jax.experimental import pallas as pl
from jax.experimental.pallas import tpu as pltpu
from jax.experimental.pallas import tpu_sc as plsc

NC = 2
NS = 16
NW = NC * NS

BATCH = 1024
NEG = 200
DIM = 64
SLOTS = NEG + 1
MAX_SLOTS_W = 13
Q = BATCH // 4
IDXC = 128


def _transpose_quarter(gb, tb, iota16, rots):
  def per_block(r, carry):
    rows = r * 16 + iota16

    for dc in range(0, DIM, 16):
      vs = [plsc.load_gather(gb, [rows, dc + rots[j]]) for j in range(16)]
      for j in range(16):
        plsc.store_scatter(tb, [dc + rots[j], rows], vs[j])
    return carry

  lax.fori_loop(0, Q // 16, per_block, 0)


def _body(idx_hbm, oe_hbm, orl_hbm, e_hbm, rl_hbm,
          out_ot, out_oh, out_orl, out_t, out_h, out_rl,
          idx_v, sidx_v, gbuf, tbuf, sem_g, sem_o):
  wid = lax.axis_index("s") * NC + lax.axis_index("c")
  parity = lax.rem(wid, 2)
  s0 = lax.div(wid, 2)
  iota16 = lax.iota(jnp.int32, 16)
  rots = [lax.rem(iota16 + j, 16) for j in range(16)]

  for jj in range(MAX_SLOTS_W):
    slot = s0 + 16 * jj

    @pl.when(slot < SLOTS)
    def _stage():
      pltpu.sync_copy(idx_hbm.at[2 + slot], idx_v.at[jj])

  n_q = 48 + 4 * (s0 <= 8).astype(jnp.int32)

  def entity_loop(tbl, out3):
    def fire(t):
      jj, q = lax.div(t, 4), lax.rem(t, 4)
      g = lax.rem(t, 3)
      base = q * Q
      pltpu.async_copy(tbl.at[idx_v.at[jj, pl.ds(base, IDXC)]],
                       gbuf.at[g, pl.ds(0, IDXC)], sem_g)
      pltpu.async_copy(tbl.at[idx_v.at[jj, pl.ds(base + IDXC, IDXC)]],
                       gbuf.at[g, pl.ds(IDXC, IDXC)], sem_g)

    def wait_gather(t):
      jj, q = lax.div(t, 4), lax.rem(t, 4)
      g = lax.rem(t, 3)
      base = q * Q
      pltpu.make_async_copy(tbl.at[idx_v.at[jj, pl.ds(base, IDXC)]],
                            gbuf.at[g, pl.ds(0, IDXC)], sem_g).wait()
      pltpu.make_async_copy(tbl.at[idx_v.at[jj, pl.ds(base + IDXC, IDXC)]],
                            gbuf.at[g, pl.ds(IDXC, IDXC)], sem_g).wait()

    fire(0)
    fire(1)

    def step(t, carry):
      jj, q = lax.div(t, 4), lax.rem(t, 4)
      slot = s0 + 16 * jj
      g = lax.rem(t, 3)
      wait_gather(t)

      @pl.when(t + 2 < n_q)
      def _fire_next():
        fire(t + 2)

      @pl.when(t >= 3)
      def _drain_out():
        pltpu.make_async_copy(tbuf.at[0], out3.at[0, :, pl.ds(0, Q)],
                              sem_o).wait()

      _transpose_quarter(gbuf.at[g], tbuf.at[g], iota16, rots)
      pltpu.async_copy(tbuf.at[g], out3.at[slot, :, pl.ds(q * Q, Q)], sem_o)
      return carry

    lax.fori_loop(0, n_q, step, 0)
    for _ in range(3):
      pltpu.make_async_copy(tbuf.at[0], out3.at[0, :, pl.ds(0, Q)],
                            sem_o).wait()

  @pl.when(parity == 0)
  def _even():
    entity_loop(oe_hbm, out_ot)

  @pl.when(parity == 1)
  def _odd():
    entity_loop(e_hbm, out_t)

  def small_job(tbl, out2, idx_row):
    pltpu.sync_copy(idx_hbm.at[idx_row], sidx_v)

    def sq(q, carry):
      base = q * Q
      c1 = pltpu.async_copy(tbl.at[sidx_v.at[pl.ds(base, IDXC)]],
                            gbuf.at[0, pl.ds(0, IDXC)], sem_g)
      c2 = pltpu.async_copy(tbl.at[sidx_v.at[pl.ds(base + IDXC, IDXC)]],
                            gbuf.at[0, pl.ds(IDXC, IDXC)], sem_g)
      c1.wait()
      c2.wait()
      _transpose_quarter(gbuf.at[0], tbuf.at[0], iota16, rots)
      pltpu.sync_copy(tbuf.at[0], out2.at[:, pl.ds(base, Q)])
      return carry

    lax.fori_loop(0, 4, sq, 0)

  @pl.when(wid == 28)
  def _oh():
    small_job(oe_hbm, out_oh, 0)

  @pl.when(wid == 29)
  def _h():
    small_job(e_hbm, out_h, 0)

  @pl.when(wid == 30)
  def _orl():
    small_job(orl_hbm, out_orl, 1)

  @pl.when(wid == 31)
  def _rl():
    small_job(rl_hbm, out_rl, 1)


@jax.jit
def _gather_all(idx_all, oe, orl, e, rl):
  mesh = plsc.VectorSubcoreMesh(core_axis_name="c", subcore_axis_name="s")
  f32 = jnp.float32
  run = functools.partial(
      pl.kernel,
      out_type=(
          jax.ShapeDtypeStruct((SLOTS, DIM, BATCH), f32),
          jax.ShapeDtypeStruct((DIM, BATCH), f32),
          jax.ShapeDtypeStruct((DIM, BATCH), f32),
          jax.ShapeDtypeStruct((SLOTS, DIM, BATCH), f32),
          jax.ShapeDtypeStruct((DIM, BATCH), f32),
          jax.ShapeDtypeStruct((DIM, BATCH), f32),
      ),
      mesh=mesh,
      compiler_params=pltpu.CompilerParams(use_tc_tiling_on_sc=False,
                                           needs_layout_passes=False),
      scratch_types=[
          pltpu.VMEM((MAX_SLOTS_W, BATCH), jnp.int32),
          pltpu.VMEM((BATCH,), jnp.int32),
          pltpu.VMEM((3, Q, DIM), f32),
          pltpu.VMEM((3, DIM, Q), f32),
          pltpu.SemaphoreType.DMA,
          pltpu.SemaphoreType.DMA,
      ],
  )(_body)
  return run(idx_all, oe, orl, e, rl)


def kernel(positive, negative, origin_entity_embedding,
           origin_relation_embedding, entity_embedding, relation_embedding):
  idx_all = jnp.concatenate([positive.T, negative.T], axis=0).astype(jnp.int32)

  ot_p, oh_p, orl_p, t_p, h_p, rl_p = _gather_all(
      idx_all, origin_entity_embedding, origin_relation_embedding,
      entity_embedding, relation_embedding)

  return (oh_p.T.reshape(BATCH, 1, DIM),
          orl_p.T.reshape(BATCH, 1, DIM),
          ot_p.transpose(2, 0, 1),
          h_p.T.reshape(BATCH, 1, DIM),
          rl_p.T.reshape(BATCH, 1, DIM),
          t_p.transpose(2, 0, 1))


# speedup vs baseline: 2.9270x; 1.2760x over previous
import functools

import jax
import jax.numpy as jnp
from jax import lax
from jax.experimental import pallas as pl
from jax.experimental.pallas import tpu as pltpu
from jax.experimental.pallas import tpu_sc as plsc

NC = 2
NS = 16
NW = NC * NS

BATCH = 1024
NEG = 200
DIM = 64
SLOTS = NEG + 1
MAX_SLOTS_W = 13
Q = BATCH // 4
IDXC = 128


def _transpose_quarter(gb, tb, iota16, rots):
  def per_block(r, carry):
    rows = r * 16 + iota16

    for dc in range(0, DIM, 16):
      vs = [plsc.load_gather(gb, [rows, dc + rots[j]]) for j in range(16)]
      for j in range(16):
        plsc.store_scatter(tb, [dc + rots[j], rows], vs[j])
    return carry

  lax.fori_loop(0, Q // 16, per_block, 0)


def _body(idx_hbm, oe_hbm, orl_hbm, e_hbm, rl_hbm,
          out_ot, out_oh, out_orl, out_t, out_h, out_rl,
          idx_v, sidx_v, gbuf, tbuf, sem_g, sem_o):
  wid = lax.axis_index("s") * NC + lax.axis_index("c")
  parity = lax.rem(wid, 2)
  s0 = lax.div(wid, 2)
  iota16 = lax.iota(jnp.int32, 16)
  rots = [lax.rem(iota16 + j, 16) for j in range(16)]

  for jj in range(MAX_SLOTS_W):
    slot = s0 + 16 * jj

    @pl.when(slot < SLOTS)
    def _stage():
      pltpu.sync_copy(idx_hbm.at[2 + slot], idx_v.at[jj])

  n_q = 48 + 4 * (s0 <= 8).astype(jnp.int32)

  def entity_loop(tbl, out3):
    def fire(t):
      jj, q = lax.div(t, 4), lax.rem(t, 4)
      g = lax.rem(t, 3)
      base = q * Q
      pltpu.async_copy(tbl.at[idx_v.at[jj, pl.ds(base, IDXC)]],
                       gbuf.at[g, pl.ds(0, IDXC)], sem_g)
      pltpu.async_copy(tbl.at[idx_v.at[jj, pl.ds(base + IDXC, IDXC)]],
                       gbuf.at[g, pl.ds(IDXC, IDXC)], sem_g)

    def wait_gather(t):
      jj, q = lax.div(t, 4), lax.rem(t, 4)
      g = lax.rem(t, 3)
      base = q * Q
      pltpu.make_async_copy(tbl.at[idx_v.at[jj, pl.ds(base, IDXC)]],
                            gbuf.at[g, pl.ds(0, IDXC)], sem_g).wait()
      pltpu.make_async_copy(tbl.at[idx_v.at[jj, pl.ds(base + IDXC, IDXC)]],
                            gbuf.at[g, pl.ds(IDXC, IDXC)], sem_g).wait()

    fire(0)
    fire(1)

    def step(t, carry):
      jj, q = lax.div(t, 4), lax.rem(t, 4)
      slot = s0 + 16 * jj
      g = lax.rem(t, 3)
      wait_gather(t)

      @pl.when(t + 2 < n_q)
      def _fire_next():
        fire(t + 2)

      @pl.when(t >= 3)
      def _drain_out():
        for _ in range(16):
          pltpu.make_async_copy(tbuf.at[0, pl.ds(0, 8), pl.ds(0, 128)],
                                out3.at[0, 0, 0], sem_o).wait()

      _transpose_quarter(gbuf.at[g], tbuf.at[g], iota16, rots)
      for r in range(8):
        for cl in range(2):
          pltpu.async_copy(tbuf.at[g, pl.ds(8 * r, 8), pl.ds(128 * cl, 128)],
                           out3.at[slot, r, 2 * q + cl], sem_o)
      return carry

    lax.fori_loop(0, n_q, step, 0)
    for _ in range(3 * 16):
      pltpu.make_async_copy(tbuf.at[0, pl.ds(0, 8), pl.ds(0, 128)],
                            out3.at[0, 0, 0], sem_o).wait()

  @pl.when(parity == 0)
  def _even():
    entity_loop(oe_hbm, out_ot)

  @pl.when(parity == 1)
  def _odd():
    entity_loop(e_hbm, out_t)

  def small_job(tbl, out2, idx_row):
    pltpu.sync_copy(idx_hbm.at[idx_row], sidx_v)

    def sq(q, carry):
      base = q * Q
      c1 = pltpu.async_copy(tbl.at[sidx_v.at[pl.ds(base, IDXC)]],
                            gbuf.at[0, pl.ds(0, IDXC)], sem_g)
      c2 = pltpu.async_copy(tbl.at[sidx_v.at[pl.ds(base + IDXC, IDXC)]],
                            gbuf.at[0, pl.ds(IDXC, IDXC)], sem_g)
      c1.wait()
      c2.wait()
      _transpose_quarter(gbuf.at[0], tbuf.at[0], iota16, rots)
      for r in range(8):
        for cl in range(2):
          pltpu.sync_copy(tbuf.at[0, pl.ds(8 * r, 8), pl.ds(128 * cl, 128)],
                          out2.at[r, 2 * q + cl])
      return carry

    lax.fori_loop(0, 4, sq, 0)

  @pl.when(wid == 28)
  def _oh():
    small_job(oe_hbm, out_oh, 0)

  @pl.when(wid == 29)
  def _h():
    small_job(e_hbm, out_h, 0)

  @pl.when(wid == 30)
  def _orl():
    small_job(orl_hbm, out_orl, 1)

  @pl.when(wid == 31)
  def _rl():
    small_job(rl_hbm, out_rl, 1)


@jax.jit
def _gather_all(idx_all, oe, orl, e, rl):
  mesh = plsc.VectorSubcoreMesh(core_axis_name="c", subcore_axis_name="s")
  f32 = jnp.float32
  run = functools.partial(
      pl.kernel,
      out_type=(
          jax.ShapeDtypeStruct((SLOTS, 8, 8, 8, 128), f32),
          jax.ShapeDtypeStruct((8, 8, 8, 128), f32),
          jax.ShapeDtypeStruct((8, 8, 8, 128), f32),
          jax.ShapeDtypeStruct((SLOTS, 8, 8, 8, 128), f32),
          jax.ShapeDtypeStruct((8, 8, 8, 128), f32),
          jax.ShapeDtypeStruct((8, 8, 8, 128), f32),
      ),
      mesh=mesh,
      compiler_params=pltpu.CompilerParams(use_tc_tiling_on_sc=False,
                                           needs_layout_passes=False),
      scratch_types=[
          pltpu.VMEM((MAX_SLOTS_W, BATCH), jnp.int32),
          pltpu.VMEM((BATCH,), jnp.int32),
          pltpu.VMEM((3, Q, DIM), f32),
          pltpu.VMEM((3, DIM, Q), f32),
          pltpu.SemaphoreType.DMA,
          pltpu.SemaphoreType.DMA,
      ],
  )(_body)
  return run(idx_all, oe, orl, e, rl)


def kernel(positive, negative, origin_entity_embedding,
           origin_relation_embedding, entity_embedding, relation_embedding):
  idx_all = jnp.concatenate([positive.T, negative.T], axis=0).astype(jnp.int32)

  ot_p, oh_p, orl_p, t_p, h_p, rl_p = _gather_all(
      idx_all, origin_entity_embedding, origin_relation_embedding,
      entity_embedding, relation_embedding)

  def untile_small(x):
    return x.transpose(1, 3, 0, 2).reshape(BATCH, 1, DIM)

  def untile_tail(x):
    return x.transpose(2, 4, 0, 1, 3).reshape(BATCH, SLOTS, DIM)

  return (untile_small(oh_p), untile_small(orl_p), untile_tail(ot_p),
          untile_small(h_p), untile_small(rl_p), untile_tail(t_p))
